# Initial kernel scaffold; baseline (speedup 1.0000x reference)
import jax
import jax.numpy as jnp
from jax.experimental import pallas as pl


def kernel(embeddings, positive_nodes, negative_nodes):
    raise NotImplementedError("write your pallas kernel here")


# speedup vs baseline: 1.2615x; 1.2615x over previous
import functools

import jax
import jax.numpy as jnp
from jax import lax
from jax.experimental import pallas as pl
from jax.experimental.pallas import tpu as pltpu
from jax.experimental.pallas import tpu_sc as plsc

NUM_ROWS = 100000
DIM = 128
NUM_IDX = 50000
EPS = 1e-8

SC_TILES = 16
SC_CHUNKS = 25
SC_LANEBLK = 128
PAD_IDX = SC_TILES * SC_CHUNKS * SC_LANEBLK

BLK = 2000
NUM_BLKS = NUM_ROWS // BLK


def _sc_histogram(idx3, val3, zeros_hbm):
  mesh = plsc.VectorSubcoreMesh(core_axis_name="c", subcore_axis_name="s")

  @functools.partial(
      pl.kernel,
      mesh=mesh,
      out_type=jax.ShapeDtypeStruct((2, NUM_ROWS), jnp.float32),
      scratch_types=[
          pltpu.VMEM((SC_CHUNKS, SC_LANEBLK), jnp.int32),
          pltpu.VMEM((SC_CHUNKS, SC_LANEBLK), jnp.float32),
          pltpu.VMEM_SHARED((NUM_ROWS,), jnp.float32),
      ],
  )
  def k(idx_hbm, val_hbm, zero_hbm, out_hbm, idx_v, val_v, shared):
    c = lax.axis_index("c")
    s = lax.axis_index("s")

    @pl.when(s == 0)
    def _():
      pltpu.sync_copy(zero_hbm, shared)

    pltpu.sync_copy(idx_hbm.at[c, s], idx_v)
    pltpu.sync_copy(val_hbm.at[s], val_v)
    plsc.subcore_barrier()

    for j in range(SC_CHUNKS):
      pltpu.sync_copy(val_v.at[j], shared.at[idx_v.at[j]], add=True)

    plsc.subcore_barrier()

    @pl.when(s == 0)
    def _():
      pltpu.sync_copy(shared, out_hbm.at[c])

  return k(idx3, val3, zeros_hbm)


def _tc_loss_body(emb_ref, cnt_ref, out_ref, acc_ref):
  i = pl.program_id(0)
  e = emb_ref[...]
  sq = e * e
  ones8 = jnp.ones((DIM, 8), jnp.float32)
  n2 = jnp.dot(sq, ones8, preferred_element_type=jnp.float32)
  r8 = lax.rsqrt(jnp.maximum(n2, EPS * EPS))
  colmask = (lax.broadcasted_iota(jnp.int32, (BLK, 8), 1) & 1) == 1
  m = jnp.where(colmask, r8, 1.0)
  x = cnt_ref[0]
  w = x * m
  part = lax.dot_general(w, e, (((0,), (0,)), ((), ())),
                         preferred_element_type=jnp.float32)

  @pl.when(i == 0)
  def _():
    acc_ref[...] = jnp.zeros_like(acc_ref)

  acc_ref[...] += part

  @pl.when(i == NUM_BLKS - 1)
  def _():
    a = acc_ref[...]
    n = jnp.float32(NUM_IDX)
    sp, tp, sn, tn = a[0], a[1], a[2], a[3]
    mcp = jnp.sum(sp * tp) / (n * jnp.maximum(jnp.sqrt(jnp.sum(sp * sp)),
                                              n * EPS))
    mcn = jnp.sum(sn * tn) / (n * jnp.maximum(jnp.sqrt(jnp.sum(sn * sn)),
                                              n * EPS))
    out_ref[...] = jnp.full((1, 1), 2.0 - mcp - mcn, jnp.float32)


def _tc_loss(embeddings, counts):
  return pl.pallas_call(
      _tc_loss_body,
      grid=(NUM_BLKS,),
      in_specs=[
          pl.BlockSpec((BLK, DIM), lambda i: (i, 0)),
          pl.BlockSpec((1, BLK, 8), lambda i: (i, 0, 0)),
      ],
      out_specs=pl.BlockSpec((1, 1), lambda i: (0, 0)),
      out_shape=jax.ShapeDtypeStruct((1, 1), jnp.float32),
      scratch_shapes=[pltpu.VMEM((8, DIM), jnp.float32)],
  )(embeddings, counts)


def kernel(embeddings, positive_nodes, negative_nodes):
  pad = PAD_IDX - NUM_IDX
  idx_p = jnp.concatenate(
      [positive_nodes.astype(jnp.int32),
       jnp.zeros((pad,), jnp.int32)]).reshape(SC_TILES, SC_CHUNKS, SC_LANEBLK)
  idx_n = jnp.concatenate(
      [negative_nodes.astype(jnp.int32),
       jnp.zeros((pad,), jnp.int32)]).reshape(SC_TILES, SC_CHUNKS, SC_LANEBLK)
  idx3 = jnp.stack([idx_p, idx_n], axis=0)
  val3 = jnp.concatenate(
      [jnp.ones((NUM_IDX,), jnp.float32),
       jnp.zeros((pad,), jnp.float32)]).reshape(SC_TILES, SC_CHUNKS,
                                                SC_LANEBLK)
  zeros_hbm = jnp.zeros((NUM_ROWS,), jnp.float32)

  counts = _sc_histogram(idx3, val3, zeros_hbm)
  c2 = counts.T
  counts_r = jnp.concatenate(
      [c2[:, (0, 0, 1, 1)],
       jnp.zeros((NUM_ROWS, 4), jnp.float32)], axis=1).reshape(
           NUM_BLKS, BLK, 8)
  loss = _tc_loss(embeddings, counts_r)
  return loss[0, 0]
